# baseline (speedup 1.0000x reference)
import functools

import jax
import jax.numpy as jnp
from jax import lax
from jax.experimental import pallas as pl
from jax.experimental.pallas import tpu as pltpu
from jax.experimental.pallas import tpu_sc as plsc

_NC = 2
_NS = 16
_NW = _NC * _NS
_ROWS = 16
_W = 4096
_L = 16
_SENT = 0x7FFFFFFF


@functools.lru_cache(maxsize=None)
def _build(n, v, d):
    npass = v // (_NW * _ROWS)
    nwin = n // _W
    posmask = (1 << 17) - 1
    assert n <= (1 << 17)

    mesh = plsc.VectorSubcoreMesh(core_axis_name="c", subcore_axis_name="s")

    @functools.partial(
        pl.kernel,
        out_type=jax.ShapeDtypeStruct((n, d), jnp.float32),
        mesh=mesh,
        scratch_types=[
            pltpu.VMEM((_ROWS, d), jnp.float32),
            pltpu.VMEM((_W,), jnp.int32),
            pltpu.VMEM((_W,), jnp.int32),
            pltpu.VMEM((_L,), jnp.int32),
            pltpu.SemaphoreType.DMA,
            pltpu.SemaphoreType.DMA,
            pltpu.SemaphoreType.DMA,
        ],
    )
    def emb(idx_hbm, table_hbm, out_hbm, cache, ib0, ib1, encbuf,
            is0, is1, wsem):
        wid = lax.axis_index("s") * _NC + lax.axis_index("c")
        lane = lax.iota(jnp.int32, _L)
        woff = wid & (nwin - 1)

        def weff_of(w):
            return (w + woff) & (nwin - 1)

        def load_win(weff, ib, sem):
            pltpu.async_copy(idx_hbm.at[pl.ds(weff * _W, _W)], ib, sem)

        def wait_win(ib, sem):
            pltpu.make_async_copy(idx_hbm.at[pl.ds(0, _W)], ib, sem).wait()

        def drain_one(j, c):
            pltpu.make_async_copy(cache.at[pl.ds(0, 1)],
                                  out_hbm.at[pl.ds(0, 1)], wsem).wait()
            return c

        def gat(vec, idx):
            return lax.gather(
                vec, idx[:, None],
                lax.GatherDimensionNumbers(
                    offset_dims=(), collapsed_slice_dims=(0,),
                    start_index_map=(0,)),
                (1,),
                mode=lax.GatherScatterMode.PROMISE_IN_BOUNDS)

        shidx = [jnp.maximum(lane - s, 0) for s in (1, 2, 4, 8)]

        def mintree(v):
            for idx in shidx:
                v = jnp.minimum(v, gat(v, idx))
            return v[_L - 1]

        def sumtree(v):
            for s, idx in zip((1, 2, 4, 8), shidx):
                v = v + jnp.where(lane >= s, gat(v, idx), 0)
            return v[_L - 1]

        def process_window(weff, ib, r0):
            wbase = weff * _W

            def scan(i, off):
                x = ib[pl.ds(i * _L, _L)]
                m = (x >= r0) & (x < r0 + _ROWS)
                pc = sumtree(jnp.where(m, 1, 0))
                enc = jnp.where(
                    m, ((x - r0) << 17) + (wbase + i * _L + lane), _SENT)
                encbuf[pl.ds(0, _L)] = enc

                def body(j, carry):
                    e = encbuf[pl.ds(0, _L)]
                    m0 = mintree(e)
                    row_local = m0 >> 17
                    pos = m0 & posmask
                    pltpu.async_copy(cache.at[pl.ds(row_local, 1)],
                                     out_hbm.at[pl.ds(pos, 1)], wsem)
                    encbuf[pl.ds(0, _L)] = jnp.where(e == m0, _SENT, e)
                    return carry

                lax.fori_loop(0, pc, body, 0)
                return off + pc

            return lax.fori_loop(0, _W // _L, scan, 0)

        def pass_body(p, carry):
            r0 = wid * (npass * _ROWS) + p * _ROWS
            pltpu.sync_copy(table_hbm.at[pl.ds(r0, _ROWS)], cache)
            load_win(weff_of(0), ib0, is0)
            load_win(weff_of(1), ib1, is1)

            def wpair(g, prev):
                w0 = 2 * g
                wait_win(ib0, is0)
                c0 = process_window(weff_of(w0), ib0, r0)
                lax.fori_loop(0, prev, drain_one, 0)

                @pl.when(g < nwin // 2 - 1)
                def _():
                    load_win(weff_of(w0 + 2), ib0, is0)

                wait_win(ib1, is1)
                c1 = process_window(weff_of(w0 + 1), ib1, r0)
                lax.fori_loop(0, c0, drain_one, 0)

                @pl.when(g < nwin // 2 - 1)
                def _():
                    load_win(weff_of(w0 + 3), ib1, is1)

                return c1

            last = lax.fori_loop(0, nwin // 2, wpair, 0)
            lax.fori_loop(0, last, drain_one, 0)
            return carry

        lax.fori_loop(0, npass, pass_body, 0)

    return emb


def kernel(indices, embedding_weight):
    b, t = indices.shape
    v, d = embedding_weight.shape
    flat = indices.reshape(-1).astype(jnp.int32)
    out = _build(flat.shape[0], v, d)(flat, embedding_weight)
    return out.reshape(b, t, d)


# speedup vs baseline: 1.5074x; 1.5074x over previous
import functools

import jax
import jax.numpy as jnp
from jax import lax
from jax.experimental import pallas as pl
from jax.experimental.pallas import tpu as pltpu
from jax.experimental.pallas import tpu_sc as plsc

_NC = 2
_NS = 16
_NW = _NC * _NS
_ROWS = 16
_W = 4096
_L = 16
_G = _L * _L
_SENT = 0x7FFFFFFF


@functools.lru_cache(maxsize=None)
def _build(n, v, d):
    npass = v // (_NW * _ROWS)
    nwin = n // _W
    posmask = (1 << 17) - 1
    assert n <= (1 << 17)

    mesh = plsc.VectorSubcoreMesh(core_axis_name="c", subcore_axis_name="s")

    @functools.partial(
        pl.kernel,
        out_type=jax.ShapeDtypeStruct((n, d), jnp.float32),
        mesh=mesh,
        scratch_types=[
            pltpu.VMEM((_ROWS, d), jnp.float32),
            pltpu.VMEM((_W,), jnp.int32),
            pltpu.VMEM((_W,), jnp.int32),
            pltpu.VMEM((_L,), jnp.int32),
            pltpu.SMEM((8,), jnp.int32),
            pltpu.SemaphoreType.DMA,
            pltpu.SemaphoreType.DMA,
            pltpu.SemaphoreType.DMA,
        ],
    )
    def emb(idx_hbm, table_hbm, out_hbm, cache, ib0, ib1, encbuf, cnt,
            is0, is1, wsem):
        wid = lax.axis_index("s") * _NC + lax.axis_index("c")
        lane = lax.iota(jnp.int32, _L)
        woff = wid & (nwin - 1)

        def weff_of(w):
            return (w + woff) & (nwin - 1)

        def load_win(weff, ib, sem):
            pltpu.async_copy(idx_hbm.at[pl.ds(weff * _W, _W)], ib, sem)

        def wait_win(ib, sem):
            pltpu.make_async_copy(idx_hbm.at[pl.ds(0, _W)], ib, sem).wait()

        def drain_one(j, c):
            pltpu.make_async_copy(cache.at[pl.ds(0, 1)],
                                  out_hbm.at[pl.ds(0, 1)], wsem).wait()
            return c

        def gat(vec, idx):
            return lax.gather(
                vec, idx[:, None],
                lax.GatherDimensionNumbers(
                    offset_dims=(), collapsed_slice_dims=(0,),
                    start_index_map=(0,)),
                (1,),
                mode=lax.GatherScatterMode.PROMISE_IN_BOUNDS)

        shidx = [jnp.maximum(lane - s, 0) for s in (1, 2, 4, 8)]

        def mintree(v):
            for idx in shidx:
                v = jnp.minimum(v, gat(v, idx))
            return v[_L - 1]

        def ortree(v):
            for idx in shidx:
                v = v | gat(v, idx)
            return v[_L - 1]

        def sumtree(v):
            for s, idx in zip((1, 2, 4, 8), shidx):
                v = v + jnp.where(lane >= s, gat(v, idx), 0)
            return v[_L - 1]

        def process_window(weff, ib, r0):
            wbase = weff * _W

            def group(h, carry):
                base = h * _G
                acc = lane * 0
                for l in range(_L):
                    x = ib[pl.ds(base + l * _L, _L)]
                    dlt = x - r0
                    m = (dlt >= 0) & (dlt < _ROWS)
                    acc = acc | jnp.where(m, 1 << l, 0)
                gbits = ortree(acc)

                for l in range(_L):
                    @pl.when(((gbits >> l) & 1) != 0)
                    def _(l=l):
                        x = ib[pl.ds(base + l * _L, _L)]
                        dlt = x - r0
                        m = (dlt >= 0) & (dlt < _ROWS)
                        pc = sumtree(jnp.where(m, 1, 0))
                        enc = jnp.where(
                            m,
                            (dlt << 17) + (wbase + base + l * _L + lane),
                            _SENT)
                        encbuf[pl.ds(0, _L)] = enc

                        def peel(j, c):
                            e = encbuf[pl.ds(0, _L)]
                            m0 = mintree(e)
                            row_local = m0 >> 17
                            pos = m0 & posmask
                            pltpu.async_copy(
                                cache.at[pl.ds(row_local, 1)],
                                out_hbm.at[pl.ds(pos, 1)], wsem)
                            encbuf[pl.ds(0, _L)] = jnp.where(
                                e == m0, _SENT, e)
                            return c

                        lax.fori_loop(0, pc, peel, 0)
                        cnt[0] = cnt[0] + pc
                return carry

            lax.fori_loop(0, _W // _G, group, 0)

        def pass_body(p, carry):
            r0 = wid * (npass * _ROWS) + p * _ROWS
            pltpu.sync_copy(table_hbm.at[pl.ds(r0, _ROWS)], cache)
            cnt[0] = 0
            load_win(weff_of(0), ib0, is0)
            load_win(weff_of(1), ib1, is1)

            def wpair(g, dt):
                drained, totprev = dt
                w0 = 2 * g
                wait_win(ib0, is0)
                process_window(weff_of(w0), ib0, r0)
                tot0 = cnt[0]
                lax.fori_loop(0, totprev - drained, drain_one, 0)

                @pl.when(g < nwin // 2 - 1)
                def _():
                    load_win(weff_of(w0 + 2), ib0, is0)

                wait_win(ib1, is1)
                process_window(weff_of(w0 + 1), ib1, r0)
                tot1 = cnt[0]
                lax.fori_loop(0, tot0 - totprev, drain_one, 0)

                @pl.when(g < nwin // 2 - 1)
                def _():
                    load_win(weff_of(w0 + 3), ib1, is1)

                return (tot0, tot1)

            drained, totprev = lax.fori_loop(0, nwin // 2, wpair, (0, 0))
            lax.fori_loop(0, cnt[0] - drained, drain_one, 0)
            return carry

        lax.fori_loop(0, npass, pass_body, 0)

    return emb


def kernel(indices, embedding_weight):
    b, t = indices.shape
    v, d = embedding_weight.shape
    flat = indices.reshape(-1).astype(jnp.int32)
    out = _build(flat.shape[0], v, d)(flat, embedding_weight)
    return out.reshape(b, t, d)


# speedup vs baseline: 1.5869x; 1.0528x over previous
import functools

import jax
import jax.numpy as jnp
from jax import lax
from jax.experimental import pallas as pl
from jax.experimental.pallas import tpu as pltpu
from jax.experimental.pallas import tpu_sc as plsc

_NC = 2
_NS = 16
_NW = _NC * _NS
_ROWS = 24
_W = 4096
_L = 16
_G = _L * _L
_SENT = 0x7FFFFFFF


@functools.lru_cache(maxsize=None)
def _build(n, v, d):
    rows_tile = v // _NW
    npass = 3
    nwin = n // _W
    posmask = (1 << 17) - 1
    assert n <= (1 << 17)

    mesh = plsc.VectorSubcoreMesh(core_axis_name="c", subcore_axis_name="s")

    @functools.partial(
        pl.kernel,
        out_type=jax.ShapeDtypeStruct((n, d), jnp.float32),
        mesh=mesh,
        scratch_types=[
            pltpu.VMEM((_ROWS, d), jnp.float32),
            pltpu.VMEM((_W,), jnp.int32),
            pltpu.VMEM((_W,), jnp.int32),
            pltpu.VMEM((_L,), jnp.int32),
            pltpu.SMEM((8,), jnp.int32),
            pltpu.SemaphoreType.DMA,
            pltpu.SemaphoreType.DMA,
            pltpu.SemaphoreType.DMA,
        ],
    )
    def emb(idx_hbm, table_hbm, out_hbm, cache, ib0, ib1, encbuf, cnt,
            is0, is1, wsem):
        wid = lax.axis_index("s") * _NC + lax.axis_index("c")
        lane = lax.iota(jnp.int32, _L)
        woff = wid & (nwin - 1)

        def weff_of(w):
            return (w + woff) & (nwin - 1)

        def load_win(weff, ib, sem):
            pltpu.async_copy(idx_hbm.at[pl.ds(weff * _W, _W)], ib, sem)

        def wait_win(ib, sem):
            pltpu.make_async_copy(idx_hbm.at[pl.ds(0, _W)], ib, sem).wait()

        def drain_one(j, c):
            pltpu.make_async_copy(cache.at[pl.ds(0, 1)],
                                  out_hbm.at[pl.ds(0, 1)], wsem).wait()
            return c

        def gat(vec, idx):
            return lax.gather(
                vec, idx[:, None],
                lax.GatherDimensionNumbers(
                    offset_dims=(), collapsed_slice_dims=(0,),
                    start_index_map=(0,)),
                (1,),
                mode=lax.GatherScatterMode.PROMISE_IN_BOUNDS)

        shidx = [jnp.maximum(lane - s, 0) for s in (1, 2, 4, 8)]

        def mintree(v):
            for idx in shidx:
                v = jnp.minimum(v, gat(v, idx))
            return v[_L - 1]

        def ortree(v):
            for idx in shidx:
                v = v | gat(v, idx)
            return v[_L - 1]

        def sumtree(v):
            for s, idx in zip((1, 2, 4, 8), shidx):
                v = v + jnp.where(lane >= s, gat(v, idx), 0)
            return v[_L - 1]

        def process_window(weff, ib, r0, r0b, rows_p):
            wbase = weff * _W

            def group(h, carry):
                base = h * _G
                acc = lane * 0
                for l in range(_L):
                    x = ib[pl.ds(base + l * _L, _L)]
                    dlt = x - r0b
                    m = (dlt >= 0) & (dlt < rows_p)
                    acc = acc | jnp.where(m, 1 << l, 0)
                gbits = ortree(acc)

                for l in range(_L):
                    @pl.when(((gbits >> l) & 1) != 0)
                    def _(l=l):
                        x = ib[pl.ds(base + l * _L, _L)]
                        dlt = x - r0b
                        m = (dlt >= 0) & (dlt < rows_p)
                        pc = sumtree(jnp.where(m, 1, 0))
                        enc = jnp.where(
                            m,
                            ((x - r0) << 17) + (wbase + base + l * _L + lane),
                            _SENT)
                        encbuf[pl.ds(0, _L)] = enc

                        def peel(j, c):
                            e = encbuf[pl.ds(0, _L)]
                            m0 = mintree(e)
                            row_local = m0 >> 17
                            pos = m0 & posmask
                            pltpu.async_copy(
                                cache.at[pl.ds(row_local, 1)],
                                out_hbm.at[pl.ds(pos, 1)], wsem)
                            encbuf[pl.ds(0, _L)] = jnp.where(
                                e == m0, _SENT, e)
                            return c

                        lax.fori_loop(0, pc, peel, 0)
                        cnt[0] = cnt[0] + pc
                return carry

            lax.fori_loop(0, _W // _G, group, 0)

        def pass_body(p, carry):
            rows_p = jnp.where(p == 2, 16, 24)
            r0 = wid * rows_tile + jnp.where(p == 2, 40, 24 * p)
            r0b = wid * rows_tile + 24 * p
            pltpu.sync_copy(table_hbm.at[pl.ds(r0, _ROWS)], cache)
            cnt[0] = 0
            load_win(weff_of(0), ib0, is0)
            load_win(weff_of(1), ib1, is1)

            def wpair(g, dt):
                drained, totprev = dt
                w0 = 2 * g
                wait_win(ib0, is0)
                process_window(weff_of(w0), ib0, r0, r0b, rows_p)
                tot0 = cnt[0]
                lax.fori_loop(0, totprev - drained, drain_one, 0)

                @pl.when(g < nwin // 2 - 1)
                def _():
                    load_win(weff_of(w0 + 2), ib0, is0)

                wait_win(ib1, is1)
                process_window(weff_of(w0 + 1), ib1, r0, r0b, rows_p)
                tot1 = cnt[0]
                lax.fori_loop(0, tot0 - totprev, drain_one, 0)

                @pl.when(g < nwin // 2 - 1)
                def _():
                    load_win(weff_of(w0 + 3), ib1, is1)

                return (tot0, tot1)

            drained, totprev = lax.fori_loop(0, nwin // 2, wpair, (0, 0))
            lax.fori_loop(0, cnt[0] - drained, drain_one, 0)
            return carry

        lax.fori_loop(0, npass, pass_body, 0)

    return emb


def kernel(indices, embedding_weight):
    b, t = indices.shape
    v, d = embedding_weight.shape
    flat = indices.reshape(-1).astype(jnp.int32)
    out = _build(flat.shape[0], v, d)(flat, embedding_weight)
    return out.reshape(b, t, d)
